# baseline (device time: 186458 ns/iter reference)
import jax
import jax.numpy as jnp
from jax import lax
from jax.experimental import pallas as pl
from jax.experimental.pallas import tpu as pltpu

N_DEV = 8
M = 2048
N = 2048
CH = M // N_DEV


def kernel(x, w_mat):
    def body(x_ref, w_ref, out_ref, sbuf, rbuf, gbuf, send_sems, recv_sems):
        p = lax.axis_index("i")
        q = p ^ ((p >> 1) & 1)
        q0 = q & 1
        q1 = (q >> 1) & 1
        q2 = (q >> 2) & 1

        def to_p(qq):
            return qq ^ ((qq >> 1) & 1)

        x_part = to_p(q ^ 1)
        y_part = to_p(q ^ 2)
        z_part = to_p(q ^ 4)

        barrier = pltpu.get_barrier_semaphore()
        for nbr in (x_part, y_part, z_part):
            pl.semaphore_signal(
                barrier, inc=1, device_id=(nbr,),
                device_id_type=pl.DeviceIdType.MESH,
            )
        pl.semaphore_wait(barrier, 3)

        out_ref[:, :] = jnp.dot(
            x_ref[:, :], w_ref[:, :], preferred_element_type=jnp.float32
        )

        keep0 = q0 * 1024
        send0 = (1 - q0) * 1024
        keep1 = keep0 + q1 * 512
        send1 = keep0 + (1 - q1) * 512
        keep2 = keep1 + q2 * 256
        send2 = keep1 + (1 - q2) * 256

        rs_stages = [
            (x_part, send0, keep0, 1024, 0, 0),
            (y_part, send1, keep1, 512, 1024, 1),
            (z_part, send2, keep2, 256, 1536, 2),
        ]
        for part, soff, koff, nrows, slot, s in rs_stages:
            sbuf[slot:slot + nrows, :] = out_ref[
                pl.ds(soff, nrows), :
            ].astype(jnp.bfloat16)
            rdma = pltpu.make_async_remote_copy(
                src_ref=sbuf.at[slot:slot + nrows, :],
                dst_ref=rbuf.at[slot:slot + nrows, :],
                send_sem=send_sems.at[s],
                recv_sem=recv_sems.at[s],
                device_id=(part,),
                device_id_type=pl.DeviceIdType.MESH,
            )
            rdma.start()
            rdma.wait()
            out_ref[pl.ds(koff, nrows), :] = (
                out_ref[pl.ds(koff, nrows), :]
                + rbuf[slot:slot + nrows, :].astype(jnp.float32)
            )

        y = out_ref[pl.ds(keep2, CH), :]
        sil = y * jax.nn.sigmoid(y)
        out_ref[pl.ds(keep2, CH), :] = sil
        gbuf[pl.ds(keep2, CH), :] = sil.astype(jnp.bfloat16)

        ag_stages = [
            (z_part, keep2, keep1 + (1 - q2) * 256, 256, 3),
            (y_part, keep1, keep0 + (1 - q1) * 512, 512, 4),
            (x_part, keep0, (1 - q0) * 1024, 1024, 5),
        ]
        for part, soff, roff, nrows, s in ag_stages:
            rdma = pltpu.make_async_remote_copy(
                src_ref=gbuf.at[pl.ds(soff, nrows), :],
                dst_ref=gbuf.at[pl.ds(soff, nrows), :],
                send_sem=send_sems.at[s],
                recv_sem=recv_sems.at[s],
                device_id=(part,),
                device_id_type=pl.DeviceIdType.MESH,
            )
            rdma.start()
            rdma.wait()
            out_ref[pl.ds(roff, nrows), :] = gbuf[
                pl.ds(roff, nrows), :
            ].astype(jnp.float32)

    return pl.pallas_call(
        body,
        out_shape=jax.ShapeDtypeStruct((M, N), jnp.float32),
        in_specs=[
            pl.BlockSpec(memory_space=pltpu.VMEM),
            pl.BlockSpec(memory_space=pltpu.VMEM),
        ],
        out_specs=pl.BlockSpec(memory_space=pltpu.VMEM),
        scratch_shapes=[
            pltpu.VMEM((1792, N), jnp.bfloat16),
            pltpu.VMEM((1792, N), jnp.bfloat16),
            pltpu.VMEM((M, N), jnp.bfloat16),
            pltpu.SemaphoreType.DMA((6,)),
            pltpu.SemaphoreType.DMA((6,)),
        ],
        compiler_params=pltpu.CompilerParams(collective_id=0),
    )(x, w_mat)


# device time: 87576 ns/iter; 2.1291x vs baseline; 2.1291x over previous
import jax
import jax.numpy as jnp
from jax import lax
from jax.experimental import pallas as pl
from jax.experimental.pallas import tpu as pltpu

N_DEV = 8
M = 2048
N = 2048
CH = M // N_DEV

COLS = ((0, 768), (768, 640), (1408, 640))
AXES_RS = ((0, 1, 2), (1, 2, 0), (2, 0, 1))
RS_SLOT = (0, 1024, 1536)


def kernel(x, w_mat):
    def body(x_ref, w_ref, out_ref, sbuf, rbuf, gbuf, send_sems, recv_sems):
        p = lax.axis_index("i")
        q = p ^ ((p >> 1) & 1)
        qb = [q & 1, (q >> 1) & 1, (q >> 2) & 1]

        def to_p(qq):
            return qq ^ ((qq >> 1) & 1)

        parts = [to_p(q ^ 1), to_p(q ^ 2), to_p(q ^ 4)]

        barrier = pltpu.get_barrier_semaphore()
        for nbr in parts:
            pl.semaphore_signal(
                barrier, inc=1, device_id=(nbr,),
                device_id_type=pl.DeviceIdType.MESH,
            )
        pl.semaphore_wait(barrier, 3)

        out_ref[:, :] = jnp.dot(
            x_ref[:, :], w_ref[:, :], preferred_element_type=jnp.float32
        )

        rs_geo = []
        keep_fin = []
        for t in range(3):
            keep = 0
            stages = []
            for j in range(3):
                bit = AXES_RS[t][j]
                nrows = 1024 >> j
                soff = keep + (1 - qb[bit]) * nrows
                keep = keep + qb[bit] * nrows
                stages.append((bit, soff, keep, nrows))
            rs_geo.append(stages)
            keep_fin.append(keep)

        def exchange(t, src_ref, soff, dst_ref, doff, nrows, bit, sem):
            c0, w = COLS[t]
            return pltpu.make_async_remote_copy(
                src_ref=src_ref.at[pl.ds(soff, nrows), pl.ds(c0, w)],
                dst_ref=dst_ref.at[pl.ds(doff, nrows), pl.ds(c0, w)],
                send_sem=send_sems.at[sem],
                recv_sem=recv_sems.at[sem],
                device_id=(parts[bit],),
                device_id_type=pl.DeviceIdType.MESH,
            )

        for j in range(3):
            rdmas = []
            for t in range(3):
                bit, soff, koff, nrows = rs_geo[t][j]
                c0, w = COLS[t]
                slot = RS_SLOT[j]
                sbuf[slot:slot + nrows, c0:c0 + w] = out_ref[
                    pl.ds(soff, nrows), c0:c0 + w
                ].astype(jnp.bfloat16)
                r = exchange(t, sbuf, slot, rbuf, slot, nrows, bit, j * 3 + t)
                r.start()
                rdmas.append(r)
            for t in range(3):
                bit, soff, koff, nrows = rs_geo[t][j]
                c0, w = COLS[t]
                slot = RS_SLOT[j]
                rdmas[t].wait()
                out_ref[pl.ds(koff, nrows), c0:c0 + w] = (
                    out_ref[pl.ds(koff, nrows), c0:c0 + w]
                    + rbuf[slot:slot + nrows, c0:c0 + w].astype(jnp.float32)
                )

        for t in range(3):
            c0, w = COLS[t]
            y = out_ref[pl.ds(keep_fin[t], CH), c0:c0 + w]
            sil = y * jax.nn.sigmoid(y)
            out_ref[pl.ds(keep_fin[t], CH), c0:c0 + w] = sil
            gbuf[pl.ds(keep_fin[t], CH), c0:c0 + w] = sil.astype(jnp.bfloat16)

        own_off = list(keep_fin)
        own_n = [CH, CH, CH]
        for j in range(3):
            rdmas = []
            recvs = []
            for t in range(3):
                bit = AXES_RS[t][2 - j]
                nrows = own_n[t]
                parent = own_off[t] - qb[bit] * nrows
                roff = parent + (1 - qb[bit]) * nrows
                r = exchange(
                    t, gbuf, own_off[t], gbuf, own_off[t], nrows, bit,
                    (3 + j) * 3 + t,
                )
                r.start()
                rdmas.append(r)
                recvs.append(roff)
                own_off[t] = parent
                own_n[t] = nrows * 2
            for t in range(3):
                c0, w = COLS[t]
                nrows = own_n[t] // 2
                rdmas[t].wait()
                out_ref[pl.ds(recvs[t], nrows), c0:c0 + w] = gbuf[
                    pl.ds(recvs[t], nrows), c0:c0 + w
                ].astype(jnp.float32)

    return pl.pallas_call(
        body,
        out_shape=jax.ShapeDtypeStruct((M, N), jnp.float32),
        in_specs=[
            pl.BlockSpec(memory_space=pltpu.VMEM),
            pl.BlockSpec(memory_space=pltpu.VMEM),
        ],
        out_specs=pl.BlockSpec(memory_space=pltpu.VMEM),
        scratch_shapes=[
            pltpu.VMEM((1792, N), jnp.bfloat16),
            pltpu.VMEM((1792, N), jnp.bfloat16),
            pltpu.VMEM((M, N), jnp.bfloat16),
            pltpu.SemaphoreType.DMA((18,)),
            pltpu.SemaphoreType.DMA((18,)),
        ],
        compiler_params=pltpu.CompilerParams(collective_id=0),
    )(x, w_mat)


# device time: 86654 ns/iter; 2.1518x vs baseline; 1.0106x over previous
import jax
import jax.numpy as jnp
from jax import lax
from jax.experimental import pallas as pl
from jax.experimental.pallas import tpu as pltpu

N_DEV = 8
M = 2048
N = 2048
CH = M // N_DEV

COLS = ((0, 768), (768, 640), (1408, 640))
AXES_RS = ((0, 1, 2), (1, 2, 0), (2, 0, 1))
RS_SLOT = (0, 1024, 1536)


def kernel(x, w_mat):
    def body(x_ref, w_ref, out_ref, abuf, rbuf, gbuf, send_sems, recv_sems):
        p = lax.axis_index("i")
        q = p ^ ((p >> 1) & 1)
        qb = [q & 1, (q >> 1) & 1, (q >> 2) & 1]

        def to_p(qq):
            return qq ^ ((qq >> 1) & 1)

        parts = [to_p(q ^ 1), to_p(q ^ 2), to_p(q ^ 4)]

        barrier = pltpu.get_barrier_semaphore()
        for nbr in parts:
            pl.semaphore_signal(
                barrier, inc=1, device_id=(nbr,),
                device_id_type=pl.DeviceIdType.MESH,
            )
        pl.semaphore_wait(barrier, 3)

        abuf[:, :] = jnp.dot(
            x_ref[:, :], w_ref[:, :], preferred_element_type=jnp.float32
        ).astype(jnp.bfloat16)

        rs_geo = []
        keep_fin = []
        for t in range(3):
            keep = 0
            stages = []
            for j in range(3):
                bit = AXES_RS[t][j]
                nrows = 1024 >> j
                soff = keep + (1 - qb[bit]) * nrows
                keep = keep + qb[bit] * nrows
                stages.append((bit, soff, keep, nrows))
            rs_geo.append(stages)
            keep_fin.append(keep)

        def exchange(t, src_ref, soff, dst_ref, doff, nrows, bit, sem):
            c0, w = COLS[t]
            return pltpu.make_async_remote_copy(
                src_ref=src_ref.at[pl.ds(soff, nrows), pl.ds(c0, w)],
                dst_ref=dst_ref.at[pl.ds(doff, nrows), pl.ds(c0, w)],
                send_sem=send_sems.at[sem],
                recv_sem=recv_sems.at[sem],
                device_id=(parts[bit],),
                device_id_type=pl.DeviceIdType.MESH,
            )

        for j in range(3):
            rdmas = []
            for t in range(3):
                bit, soff, koff, nrows = rs_geo[t][j]
                r = exchange(
                    t, abuf, soff, rbuf, RS_SLOT[j], nrows, bit, j * 3 + t
                )
                r.start()
                rdmas.append(r)
            for t in range(3):
                bit, soff, koff, nrows = rs_geo[t][j]
                c0, w = COLS[t]
                slot = RS_SLOT[j]
                rdmas[t].wait()
                abuf[pl.ds(koff, nrows), c0:c0 + w] = (
                    abuf[pl.ds(koff, nrows), c0:c0 + w]
                    + rbuf[slot:slot + nrows, c0:c0 + w]
                )

        for t in range(3):
            c0, w = COLS[t]
            y = abuf[pl.ds(keep_fin[t], CH), c0:c0 + w].astype(jnp.float32)
            sil = y * jax.nn.sigmoid(y)
            out_ref[pl.ds(keep_fin[t], CH), c0:c0 + w] = sil
            gbuf[pl.ds(keep_fin[t], CH), c0:c0 + w] = sil.astype(jnp.bfloat16)

        own_off = list(keep_fin)
        own_n = [CH, CH, CH]
        for j in range(3):
            rdmas = []
            recvs = []
            for t in range(3):
                bit = AXES_RS[t][2 - j]
                nrows = own_n[t]
                parent = own_off[t] - qb[bit] * nrows
                roff = parent + (1 - qb[bit]) * nrows
                r = exchange(
                    t, gbuf, own_off[t], gbuf, own_off[t], nrows, bit,
                    (3 + j) * 3 + t,
                )
                r.start()
                rdmas.append(r)
                recvs.append(roff)
                own_off[t] = parent
                own_n[t] = nrows * 2
            for t in range(3):
                c0, w = COLS[t]
                nrows = own_n[t] // 2
                rdmas[t].wait()
                out_ref[pl.ds(recvs[t], nrows), c0:c0 + w] = gbuf[
                    pl.ds(recvs[t], nrows), c0:c0 + w
                ].astype(jnp.float32)

    return pl.pallas_call(
        body,
        out_shape=jax.ShapeDtypeStruct((M, N), jnp.float32),
        in_specs=[
            pl.BlockSpec(memory_space=pltpu.VMEM),
            pl.BlockSpec(memory_space=pltpu.VMEM),
        ],
        out_specs=pl.BlockSpec(memory_space=pltpu.VMEM),
        scratch_shapes=[
            pltpu.VMEM((M, N), jnp.bfloat16),
            pltpu.VMEM((1792, N), jnp.bfloat16),
            pltpu.VMEM((M, N), jnp.bfloat16),
            pltpu.SemaphoreType.DMA((18,)),
            pltpu.SemaphoreType.DMA((18,)),
        ],
        compiler_params=pltpu.CompilerParams(collective_id=0),
    )(x, w_mat)


# device time: 80371 ns/iter; 2.3200x vs baseline; 1.0782x over previous
import jax
import jax.numpy as jnp
from jax import lax
from jax.experimental import pallas as pl
from jax.experimental.pallas import tpu as pltpu

N_DEV = 8
M = 2048
N = 2048
CH = N // N_DEV

HALVES = (
    ((0, 352), (352, 336)),
    ((688, 336), (1024, 336)),
    ((1360, 352), (1712, 336)),
)
AXES_RS = ((0, 1, 2), (1, 2, 0), (2, 0, 1))
CSLOT = (0, 1024, 1536)


def kernel(x, w_mat):
    def body(x_ref, w_ref, out_ref, abuf, rbuf, gbuf, send_sems, recv_sems):
        p = lax.axis_index("i")
        q = p ^ ((p >> 1) & 1)
        qb = [q & 1, (q >> 1) & 1, (q >> 2) & 1]

        def to_p(qq):
            return qq ^ ((qq >> 1) & 1)

        parts = [to_p(q ^ 1), to_p(q ^ 2), to_p(q ^ 4)]

        barrier = pltpu.get_barrier_semaphore()
        for nbr in parts:
            pl.semaphore_signal(
                barrier, inc=1, device_id=(nbr,),
                device_id_type=pl.DeviceIdType.MESH,
            )
        pl.semaphore_wait(barrier, 3)

        abuf[:, :] = jnp.dot(
            x_ref[:, :], w_ref[:, :], preferred_element_type=jnp.float32
        ).astype(jnp.bfloat16)

        ckeep = [0, 0, 0]
        rs_cols = []
        for j in range(3):
            row = []
            for t in range(3):
                bit = AXES_RS[t][j]
                ncols = 1024 >> j
                csend = ckeep[t] + (1 - qb[bit]) * ncols
                ckeep[t] = ckeep[t] + qb[bit] * ncols
                row.append((bit, csend, ckeep[t], ncols))
            rs_cols.append(row)

        def rcopy(src_ref, r0, nr, csrc, cdst, ncols, bit, sem):
            return pltpu.make_async_remote_copy(
                src_ref=src_ref.at[pl.ds(r0, nr), pl.ds(csrc, ncols)],
                dst_ref=(rbuf if src_ref is abuf else gbuf).at[
                    pl.ds(r0, nr), pl.ds(cdst, ncols)
                ],
                send_sem=send_sems.at[sem],
                recv_sem=recv_sems.at[sem],
                device_id=(parts[bit],),
                device_id_type=pl.DeviceIdType.MESH,
            )

        for j in range(3):
            rd = {}
            for h in range(2):
                for t in range(3):
                    bit, csend, ck, ncols = rs_cols[j][t]
                    r0, nr = HALVES[t][h]
                    sem = (j * 3 + t) * 2 + h
                    r = rcopy(abuf, r0, nr, csend, CSLOT[j], ncols, bit, sem)
                    r.start()
                    rd[(t, h)] = r
            for h in range(2):
                for t in range(3):
                    bit, csend, ck, ncols = rs_cols[j][t]
                    r0, nr = HALVES[t][h]
                    rd[(t, h)].wait()
                    abuf[pl.ds(r0, nr), pl.ds(ck, ncols)] = (
                        abuf[pl.ds(r0, nr), pl.ds(ck, ncols)]
                        + rbuf[pl.ds(r0, nr), pl.ds(CSLOT[j], ncols)]
                    )

        for t in range(3):
            for h in range(2):
                r0, nr = HALVES[t][h]
                y = abuf[pl.ds(r0, nr), pl.ds(ckeep[t], CH)].astype(
                    jnp.float32
                )
                sil = y * jax.nn.sigmoid(y)
                out_ref[pl.ds(r0, nr), pl.ds(ckeep[t], CH)] = sil
                gbuf[pl.ds(r0, nr), pl.ds(ckeep[t], CH)] = sil.astype(
                    jnp.bfloat16
                )

        own = [(ckeep[t], CH) for t in range(3)]
        for j in range(3):
            rd = {}
            recvs = {}
            new_own = list(own)
            for h in range(2):
                for t in range(3):
                    bit = AXES_RS[t][2 - j]
                    co, nc = own[t]
                    parent = co - qb[bit] * nc
                    r0, nr = HALVES[t][h]
                    sem = ((3 + j) * 3 + t) * 2 + h
                    r = rcopy(gbuf, r0, nr, co, co, nc, bit, sem)
                    r.start()
                    rd[(t, h)] = r
                    recvs[t] = (parent + (1 - qb[bit]) * nc, nc)
                    new_own[t] = (parent, nc * 2)
            for h in range(2):
                for t in range(3):
                    rco, nc = recvs[t]
                    r0, nr = HALVES[t][h]
                    rd[(t, h)].wait()
                    out_ref[pl.ds(r0, nr), pl.ds(rco, nc)] = gbuf[
                        pl.ds(r0, nr), pl.ds(rco, nc)
                    ].astype(jnp.float32)
            own = new_own

    return pl.pallas_call(
        body,
        out_shape=jax.ShapeDtypeStruct((M, N), jnp.float32),
        in_specs=[
            pl.BlockSpec(memory_space=pltpu.VMEM),
            pl.BlockSpec(memory_space=pltpu.VMEM),
        ],
        out_specs=pl.BlockSpec(memory_space=pltpu.VMEM),
        scratch_shapes=[
            pltpu.VMEM((M, N), jnp.bfloat16),
            pltpu.VMEM((M, 1792), jnp.bfloat16),
            pltpu.VMEM((M, N), jnp.bfloat16),
            pltpu.SemaphoreType.DMA((36,)),
            pltpu.SemaphoreType.DMA((36,)),
        ],
        compiler_params=pltpu.CompilerParams(collective_id=0),
    )(x, w_mat)


# device time: 69266 ns/iter; 2.6919x vs baseline; 1.1603x over previous
import jax
import jax.numpy as jnp
from jax import lax
from jax.experimental import pallas as pl
from jax.experimental.pallas import tpu as pltpu

N_DEV = 8
M = 2048
N = 2048
CH = N // N_DEV

HALVES = (
    ((0, 352), (352, 336)),
    ((688, 336), (1024, 336)),
    ((1360, 352), (1712, 336)),
)
AXES_RS = ((0, 1, 2), (1, 2, 0), (2, 0, 1))
CSLOT = (0, 1024, 1536)


def kernel(x, w_mat):
    def body(x_ref, w_ref, out_ref, abuf, rbuf, gbuf, send_sems, recv_sems):
        p = lax.axis_index("i")
        q = p ^ ((p >> 1) & 1)
        qb = [q & 1, (q >> 1) & 1, (q >> 2) & 1]

        def to_p(qq):
            return qq ^ ((qq >> 1) & 1)

        parts = [to_p(q ^ 1), to_p(q ^ 2), to_p(q ^ 4)]

        barrier = pltpu.get_barrier_semaphore()
        for nbr in parts:
            pl.semaphore_signal(
                barrier, inc=1, device_id=(nbr,),
                device_id_type=pl.DeviceIdType.MESH,
            )
        pl.semaphore_wait(barrier, 3)

        ck = [0, 0, 0]
        rs_cols = []
        for j in range(3):
            row = []
            for t in range(3):
                bit = AXES_RS[t][j]
                ncols = 1024 >> j
                csend = ck[t] + (1 - qb[bit]) * ncols
                ck[t] = ck[t] + qb[bit] * ncols
                row.append((bit, csend, ck[t], ncols))
            rs_cols.append(row)

        def rcopy(src_ref, dst_ref, r0, nr, csrc, cdst, ncols, bit, sem):
            return pltpu.make_async_remote_copy(
                src_ref=src_ref.at[pl.ds(r0, nr), pl.ds(csrc, ncols)],
                dst_ref=dst_ref.at[pl.ds(r0, nr), pl.ds(cdst, ncols)],
                send_sem=send_sems.at[sem],
                recv_sem=recv_sems.at[sem],
                device_id=(parts[bit],),
                device_id_type=pl.DeviceIdType.MESH,
            )

        def rs_send(j, t, h):
            bit, csend, _, ncols = rs_cols[j][t]
            r0, nr = HALVES[t][h]
            sem = (j * 3 + t) * 2 + h
            r = rcopy(abuf, rbuf, r0, nr, csend, CSLOT[j], ncols, bit, sem)
            r.start()
            return r

        def gemm_tile(r0, nr, c0, nc):
            abuf[pl.ds(r0, nr), pl.ds(c0, nc)] = jnp.dot(
                x_ref[pl.ds(r0, nr), :],
                w_ref[:, pl.ds(c0, nc)],
                preferred_element_type=jnp.float32,
            ).astype(jnp.bfloat16)

        rd = {}
        for h in range(2):
            for t in range(3):
                _, csend, ckeep0, _ = rs_cols[0][t]
                r0, nr = HALVES[t][h]
                gemm_tile(r0, nr, csend, 1024)
                rd[(0, t, h)] = rs_send(0, t, h)
        for h in range(2):
            for t in range(3):
                _, _, ckeep0, _ = rs_cols[0][t]
                r0, nr = HALVES[t][h]
                gemm_tile(r0, nr, ckeep0, 1024)

        for j in range(3):
            for h in range(2):
                for t in range(3):
                    bit, csend, ckj, ncols = rs_cols[j][t]
                    r0, nr = HALVES[t][h]
                    rd[(j, t, h)].wait()
                    abuf[pl.ds(r0, nr), pl.ds(ckj, ncols)] = (
                        abuf[pl.ds(r0, nr), pl.ds(ckj, ncols)]
                        + rbuf[pl.ds(r0, nr), pl.ds(CSLOT[j], ncols)]
                    )
                    if j < 2:
                        rd[(j + 1, t, h)] = rs_send(j + 1, t, h)
                    else:
                        y = abuf[
                            pl.ds(r0, nr), pl.ds(ck[t], CH)
                        ].astype(jnp.float32)
                        sil = y * jax.nn.sigmoid(y)
                        out_ref[pl.ds(r0, nr), pl.ds(ck[t], CH)] = sil
                        gbuf[pl.ds(r0, nr), pl.ds(ck[t], CH)] = sil.astype(
                            jnp.bfloat16
                        )
                        bit0 = AXES_RS[t][2]
                        sem = (3 * 3 + t) * 2 + h
                        r = rcopy(
                            gbuf, gbuf, r0, nr, ck[t], ck[t], CH, bit0, sem
                        )
                        r.start()
                        rd[(3, t, h)] = r

        own = [(ck[t], CH) for t in range(3)]
        for j in range(3):
            new_own = list(own)
            for h in range(2):
                for t in range(3):
                    bit = AXES_RS[t][2 - j]
                    co, nc = own[t]
                    parent = co - qb[bit] * nc
                    rco = parent + (1 - qb[bit]) * nc
                    r0, nr = HALVES[t][h]
                    rd[(3 + j, t, h)].wait()
                    out_ref[pl.ds(r0, nr), pl.ds(rco, nc)] = gbuf[
                        pl.ds(r0, nr), pl.ds(rco, nc)
                    ].astype(jnp.float32)
                    new_own[t] = (parent, nc * 2)
                    if j < 2:
                        bitn = AXES_RS[t][2 - (j + 1)]
                        sem = ((3 + j + 1) * 3 + t) * 2 + h
                        r = rcopy(
                            gbuf, gbuf, r0, nr, parent, parent,
                            nc * 2, bitn, sem,
                        )
                        r.start()
                        rd[(3 + j + 1, t, h)] = r
            own = new_own

    return pl.pallas_call(
        body,
        out_shape=jax.ShapeDtypeStruct((M, N), jnp.float32),
        in_specs=[
            pl.BlockSpec(memory_space=pltpu.VMEM),
            pl.BlockSpec(memory_space=pltpu.VMEM),
        ],
        out_specs=pl.BlockSpec(memory_space=pltpu.VMEM),
        scratch_shapes=[
            pltpu.VMEM((M, N), jnp.bfloat16),
            pltpu.VMEM((M, 1792), jnp.bfloat16),
            pltpu.VMEM((M, N), jnp.bfloat16),
            pltpu.SemaphoreType.DMA((36,)),
            pltpu.SemaphoreType.DMA((36,)),
        ],
        compiler_params=pltpu.CompilerParams(collective_id=0),
    )(x, w_mat)


# device time: 68724 ns/iter; 2.7131x vs baseline; 1.0079x over previous
import jax
import jax.numpy as jnp
from jax import lax
from jax.experimental import pallas as pl
from jax.experimental.pallas import tpu as pltpu

N_DEV = 8
M = 2048
N = 2048
CH = N // N_DEV

NS = 4
HALVES = (
    ((0, 176), (176, 176), (352, 168), (520, 168)),
    ((688, 168), (856, 168), (1024, 168), (1192, 168)),
    ((1360, 176), (1536, 176), (1712, 168), (1880, 168)),
)
AXES_RS = ((0, 1, 2), (1, 2, 0), (2, 0, 1))
CSLOT = (0, 1024, 1536)


def kernel(x, w_mat):
    def body(x_ref, w_ref, out_ref, abuf, rbuf, gbuf, send_sems, recv_sems):
        p = lax.axis_index("i")
        q = p ^ ((p >> 1) & 1)
        qb = [q & 1, (q >> 1) & 1, (q >> 2) & 1]

        def to_p(qq):
            return qq ^ ((qq >> 1) & 1)

        parts = [to_p(q ^ 1), to_p(q ^ 2), to_p(q ^ 4)]

        barrier = pltpu.get_barrier_semaphore()
        for nbr in parts:
            pl.semaphore_signal(
                barrier, inc=1, device_id=(nbr,),
                device_id_type=pl.DeviceIdType.MESH,
            )
        pl.semaphore_wait(barrier, 3)

        ck = [0, 0, 0]
        rs_cols = []
        for j in range(3):
            row = []
            for t in range(3):
                bit = AXES_RS[t][j]
                ncols = 1024 >> j
                csend = ck[t] + (1 - qb[bit]) * ncols
                ck[t] = ck[t] + qb[bit] * ncols
                row.append((bit, csend, ck[t], ncols))
            rs_cols.append(row)

        def rcopy(src_ref, dst_ref, r0, nr, csrc, cdst, ncols, bit, sem):
            return pltpu.make_async_remote_copy(
                src_ref=src_ref.at[pl.ds(r0, nr), pl.ds(csrc, ncols)],
                dst_ref=dst_ref.at[pl.ds(r0, nr), pl.ds(cdst, ncols)],
                send_sem=send_sems.at[sem],
                recv_sem=recv_sems.at[sem],
                device_id=(parts[bit],),
                device_id_type=pl.DeviceIdType.MESH,
            )

        def rs_send(j, t, h):
            bit, csend, _, ncols = rs_cols[j][t]
            r0, nr = HALVES[t][h]
            sem = (j * 3 + t) * NS + h
            r = rcopy(abuf, rbuf, r0, nr, csend, CSLOT[j], ncols, bit, sem)
            r.start()
            return r

        def gemm_tile(r0, nr, c0, nc):
            abuf[pl.ds(r0, nr), pl.ds(c0, nc)] = jnp.dot(
                x_ref[pl.ds(r0, nr), :],
                w_ref[:, pl.ds(c0, nc)],
                preferred_element_type=jnp.float32,
            ).astype(jnp.bfloat16)

        rd = {}
        for h in range(NS):
            for t in range(3):
                _, csend, ckeep0, _ = rs_cols[0][t]
                r0, nr = HALVES[t][h]
                gemm_tile(r0, nr, csend, 1024)
                rd[(0, t, h)] = rs_send(0, t, h)
        for h in range(NS):
            for t in range(3):
                _, _, ckeep0, _ = rs_cols[0][t]
                r0, nr = HALVES[t][h]
                gemm_tile(r0, nr, ckeep0, 1024)

        for j in range(3):
            for h in range(NS):
                for t in range(3):
                    bit, csend, ckj, ncols = rs_cols[j][t]
                    r0, nr = HALVES[t][h]
                    rd[(j, t, h)].wait()
                    abuf[pl.ds(r0, nr), pl.ds(ckj, ncols)] = (
                        abuf[pl.ds(r0, nr), pl.ds(ckj, ncols)]
                        + rbuf[pl.ds(r0, nr), pl.ds(CSLOT[j], ncols)]
                    )
                    if j < 2:
                        rd[(j + 1, t, h)] = rs_send(j + 1, t, h)
                    else:
                        y = abuf[
                            pl.ds(r0, nr), pl.ds(ck[t], CH)
                        ].astype(jnp.float32)
                        sil = y * jax.nn.sigmoid(y)
                        out_ref[pl.ds(r0, nr), pl.ds(ck[t], CH)] = sil
                        gbuf[pl.ds(r0, nr), pl.ds(ck[t], CH)] = sil.astype(
                            jnp.bfloat16
                        )
                        bit0 = AXES_RS[t][2]
                        sem = (3 * 3 + t) * NS + h
                        r = rcopy(
                            gbuf, gbuf, r0, nr, ck[t], ck[t], CH, bit0, sem
                        )
                        r.start()
                        rd[(3, t, h)] = r

        own = [(ck[t], CH) for t in range(3)]
        for j in range(3):
            new_own = list(own)
            for h in range(NS):
                for t in range(3):
                    bit = AXES_RS[t][2 - j]
                    co, nc = own[t]
                    parent = co - qb[bit] * nc
                    rco = parent + (1 - qb[bit]) * nc
                    r0, nr = HALVES[t][h]
                    rd[(3 + j, t, h)].wait()
                    out_ref[pl.ds(r0, nr), pl.ds(rco, nc)] = gbuf[
                        pl.ds(r0, nr), pl.ds(rco, nc)
                    ].astype(jnp.float32)
                    new_own[t] = (parent, nc * 2)
                    if j < 2:
                        bitn = AXES_RS[t][2 - (j + 1)]
                        sem = ((3 + j + 1) * 3 + t) * NS + h
                        r = rcopy(
                            gbuf, gbuf, r0, nr, parent, parent,
                            nc * 2, bitn, sem,
                        )
                        r.start()
                        rd[(3 + j + 1, t, h)] = r
            own = new_own

    return pl.pallas_call(
        body,
        out_shape=jax.ShapeDtypeStruct((M, N), jnp.float32),
        in_specs=[
            pl.BlockSpec(memory_space=pltpu.VMEM),
            pl.BlockSpec(memory_space=pltpu.VMEM),
        ],
        out_specs=pl.BlockSpec(memory_space=pltpu.VMEM),
        scratch_shapes=[
            pltpu.VMEM((M, N), jnp.bfloat16),
            pltpu.VMEM((M, 1792), jnp.bfloat16),
            pltpu.VMEM((M, N), jnp.bfloat16),
            pltpu.SemaphoreType.DMA((18 * NS,)),
            pltpu.SemaphoreType.DMA((18 * NS,)),
        ],
        compiler_params=pltpu.CompilerParams(collective_id=0),
    )(x, w_mat)
